# baseline (speedup 1.0000x reference)
import functools

import jax
import jax.numpy as jnp
from jax import lax
from jax.experimental import pallas as pl
from jax.experimental.pallas import tpu as pltpu
from jax.experimental.pallas import tpu_sc as plsc

N_NODES = 10000
NPAD = 10112
NC, NS, L = 2, 16, 16
NW = NC * NS
RPS = NPAD // NS


def _make_spmm(d, cpw, k):
    mesh = plsc.VectorSubcoreMesh(core_axis_name="c", subcore_axis_name="s")

    half = cpw // 2

    @functools.partial(
        pl.kernel,
        out_type=jax.ShapeDtypeStruct((NC, NPAD, d), jnp.float32),
        mesh=mesh,
        compiler_params=pltpu.CompilerParams(use_tc_tiling_on_sc=False),
        scratch_types=[
            pltpu.VMEM((half, k), jnp.int32),
            pltpu.VMEM((half, k), jnp.int32),
            pltpu.VMEM((3, k, d), jnp.float32),
            pltpu.VMEM_SHARED((NPAD, d), jnp.float32),
            pltpu.SemaphoreType.DMA,
            pltpu.SemaphoreType.DMA,
            pltpu.SemaphoreType.DMA,
        ],
    )
    def spmm(y_hbm, src_hbm, dst_hbm, out_hbm, sidx, didx, rows, acc,
             g0, g1, g2):
        c = lax.axis_index("c")
        s = lax.axis_index("s")
        wid = c * NS + s

        zero = jnp.zeros((L,), jnp.float32)

        def zero_row(kk, _):
            for l in range(d // L):
                rows[0, kk, pl.ds(l * L, L)] = zero
            return 0

        lax.fori_loop(0, k, zero_row, 0)
        base = s * RPS
        off = 0
        while off < RPS:
            sz = min(k, RPS - off)
            pltpu.sync_copy(rows.at[0, pl.ds(0, sz)], acc.at[pl.ds(base + off, sz)])
            off += sz
        plsc.subcore_barrier()

        gsem = (g0, g1, g2)

        def gather(j, b):
            pltpu.async_copy(y_hbm.at[sidx.at[j]], rows.at[b], gsem[b])

        def wait_gather(j, b):
            pltpu.make_async_copy(y_hbm.at[sidx.at[j]], rows.at[b], gsem[b]).wait()

        def step(i, _):
            for b in range(3):
                j = i * 3 + b
                wait_gather(j, b)

                @pl.when(j + 2 < half)
                def _():
                    gather(j + 2, (b + 2) % 3)

                pltpu.sync_copy(rows.at[b], acc.at[didx.at[j]], add=True)
            return 0

        for h in range(2):
            pltpu.sync_copy(src_hbm.at[wid, pl.ds(h * half, half)], sidx)
            pltpu.sync_copy(dst_hbm.at[wid, pl.ds(h * half, half)], didx)
            gather(0, 0)
            gather(1, 1)
            lax.fori_loop(0, half // 3, step, 0)
        plsc.subcore_barrier()
        pltpu.sync_copy(acc.at[pl.ds(base, RPS)], out_hbm.at[c, pl.ds(base, RPS)])

    return spmm


_GRID = 8
_RB = NPAD // _GRID


def _mm_body(x_ref, w_ref, o_ref):
    o_ref[:] = jnp.dot(x_ref[:], w_ref[:], preferred_element_type=jnp.float32)


def _mm(x, w):
    dn = w.shape[1]
    return pl.pallas_call(
        _mm_body,
        grid=(_GRID,),
        in_specs=[
            pl.BlockSpec((_RB, x.shape[1]), lambda i: (i, 0)),
            pl.BlockSpec(w.shape, lambda i: (0, 0)),
        ],
        out_specs=pl.BlockSpec((_RB, dn), lambda i: (i, 0)),
        out_shape=jax.ShapeDtypeStruct((NPAD, dn), jnp.float32),
    )(x, w)


def _dinv_scale_body(p_ref, xw_ref, v_ref, y_ref):
    deg = p_ref[0, :, 0:1] + p_ref[1, :, 0:1] + 1.0
    v = lax.rsqrt(deg)
    v_ref[:] = v
    y_ref[:] = xw_ref[:] * v


def _dinv_scale(partials16, xw):
    d = xw.shape[1]
    return pl.pallas_call(
        _dinv_scale_body,
        grid=(_GRID,),
        in_specs=[
            pl.BlockSpec((NC, _RB, 16), lambda i: (0, i, 0)),
            pl.BlockSpec((_RB, d), lambda i: (i, 0)),
        ],
        out_specs=[
            pl.BlockSpec((_RB, 1), lambda i: (i, 0)),
            pl.BlockSpec((_RB, d), lambda i: (i, 0)),
        ],
        out_shape=[
            jax.ShapeDtypeStruct((NPAD, 1), jnp.float32),
            jax.ShapeDtypeStruct((NPAD, d), jnp.float32),
        ],
    )(partials16, xw)


def _layer_body(p0_ref, p1_ref, y_ref, v_ref, b_ref, w_ref, o_ref):
    t = (p0_ref[:] + p1_ref[:] + y_ref[:]) * v_ref[:] + b_ref[:]
    h = jnp.maximum(t, 0.0)
    o_ref[:] = jnp.dot(h, w_ref[:], preferred_element_type=jnp.float32) * v_ref[:]


def _layer(p0, p1, y, dinv, b, w):
    d, dn = w.shape
    return pl.pallas_call(
        _layer_body,
        grid=(_GRID,),
        in_specs=[
            pl.BlockSpec((_RB, d), lambda i: (i, 0)),
            pl.BlockSpec((_RB, d), lambda i: (i, 0)),
            pl.BlockSpec((_RB, d), lambda i: (i, 0)),
            pl.BlockSpec((_RB, 1), lambda i: (i, 0)),
            pl.BlockSpec((1, d), lambda i: (0, 0)),
            pl.BlockSpec((d, dn), lambda i: (0, 0)),
        ],
        out_specs=pl.BlockSpec((_RB, dn), lambda i: (i, 0)),
        out_shape=jax.ShapeDtypeStruct((NPAD, dn), jnp.float32),
    )(p0, p1, y, dinv, b, w)


def _final_body(p0_ref, p1_ref, y_ref, v_ref, b_ref, o_ref):
    o = (p0_ref[:] + p1_ref[:] + y_ref[:]) * v_ref[:] + b_ref[:]
    z = o - jnp.max(o, axis=1, keepdims=True)
    o_ref[:] = z - jnp.log(jnp.sum(jnp.exp(z), axis=1, keepdims=True))


def _final(p0, p1, y, dinv, b):
    d = y.shape[1]
    return pl.pallas_call(
        _final_body,
        grid=(_GRID,),
        in_specs=[
            pl.BlockSpec((_RB, d), lambda i: (i, 0)),
            pl.BlockSpec((_RB, d), lambda i: (i, 0)),
            pl.BlockSpec((_RB, d), lambda i: (i, 0)),
            pl.BlockSpec((_RB, 1), lambda i: (i, 0)),
            pl.BlockSpec((1, d), lambda i: (0, 0)),
        ],
        out_specs=pl.BlockSpec((_RB, d), lambda i: (i, 0)),
        out_shape=jax.ShapeDtypeStruct((NPAD, d), jnp.float32),
    )(p0, p1, y, dinv, b)


def kernel(x, edge_index, W1, b1, W2, b2, W3, b3):
    n = x.shape[0]
    e = edge_index.shape[1]
    ei = edge_index.astype(jnp.int32)

    def edge_layout(k):
        cpw = -(-e // (NW * k))
        cpw = -(-cpw // 6) * 6
        pad = jnp.full((cpw * k * NW - e,), n, jnp.int32)
        src = jnp.concatenate([ei[0], pad]).reshape(NW, cpw, k)
        dst = jnp.concatenate([ei[1], pad]).reshape(NW, cpw, k)
        return cpw, src, dst

    cpw_a, src_a, dst_a = edge_layout(128)
    cpw_b, src_b, dst_b = edge_layout(96)
    spmm16 = _make_spmm(16, cpw_a, 128)
    spmm128 = _make_spmm(128, cpw_b, 96)
    spmm64 = _make_spmm(64, cpw_a, 128)

    ones16 = jnp.zeros((NPAD, 16), jnp.float32).at[:n, 0].set(1.0)
    deg_p = spmm16(ones16, src_a, dst_a)
    xp = jnp.zeros((NPAD, x.shape[1]), jnp.float32).at[:n].set(x)
    xw1 = _mm(xp, W1)
    dinv, y1 = _dinv_scale(deg_p, xw1)
    p1 = spmm128(y1, src_b, dst_b)
    y2 = _layer(p1[0], p1[1], y1, dinv, b1.reshape(1, -1), W2)
    p2 = spmm128(y2, src_b, dst_b)
    y3 = _layer(p2[0], p2[1], y2, dinv, b2.reshape(1, -1), W3)
    p3 = spmm64(y3, src_a, dst_a)
    out = _final(p3[0], p3[1], y3, dinv, b3.reshape(1, -1))
    return out[:n]


# speedup vs baseline: 1.7935x; 1.7935x over previous
import functools

import jax
import jax.numpy as jnp
from jax import lax
from jax.experimental import pallas as pl
from jax.experimental.pallas import tpu as pltpu
from jax.experimental.pallas import tpu_sc as plsc

N_NODES = 10000
NPAD = 10112
NC, NS, L = 2, 16, 16
NW = NC * NS
RPS = NPAD // NS


def _make_spmm(d, k, ca, cb):
    mesh = plsc.VectorSubcoreMesh(core_axis_name="c", subcore_axis_name="s")
    ha, hb = ca // 2, cb // 2

    @functools.partial(
        pl.kernel,
        out_type=jax.ShapeDtypeStruct((NC, NPAD, d), jnp.float32),
        mesh=mesh,
        compiler_params=pltpu.CompilerParams(use_tc_tiling_on_sc=False),
        scratch_types=[
            pltpu.VMEM((ha, k), jnp.int32),
            pltpu.VMEM((ha, k), jnp.int32),
            pltpu.VMEM((2, k, d), jnp.float32),
            pltpu.VMEM_SHARED((NPAD, d), jnp.float32),
            pltpu.SemaphoreType.DMA,
            pltpu.SemaphoreType.DMA,
        ],
    )
    def spmm(y_hbm, src_hbm, dst_hbm, out_hbm, sidx, didx, rows, acc, g0, g1):
        c = lax.axis_index("c")
        s = lax.axis_index("s")

        zero = jnp.zeros((L,), jnp.float32)

        def zero_row(kk, _):
            for l in range(d // L):
                rows[0, kk, pl.ds(l * L, L)] = zero
            return 0

        lax.fori_loop(0, k, zero_row, 0)
        base = s * RPS
        off = 0
        while off < RPS:
            sz = min(k, RPS - off)
            pltpu.sync_copy(rows.at[0, pl.ds(0, sz)], acc.at[pl.ds(base + off, sz)])
            off += sz
        plsc.subcore_barrier()

        gsem = (g0, g1)

        def gather(j, b):
            pltpu.async_copy(y_hbm.at[sidx.at[j]], rows.at[b], gsem[b])

        def wait_gather(j, b):
            pltpu.make_async_copy(y_hbm.at[sidx.at[j]], rows.at[b], gsem[b]).wait()

        def run(chunk_base, hh):
            def step(i, _):
                j = i * 2
                gather(j + 1, 1)
                wait_gather(j, 0)
                pltpu.sync_copy(rows.at[0], acc.at[didx.at[j]], add=True)

                @pl.when(j + 2 < hh)
                def _():
                    gather(j + 2, 0)

                wait_gather(j + 1, 1)
                pltpu.sync_copy(rows.at[1], acc.at[didx.at[j + 1]], add=True)
                return 0

            for h in range(2):
                pltpu.sync_copy(src_hbm.at[pl.ds(chunk_base + h * hh, hh)],
                                sidx.at[pl.ds(0, hh)])
                pltpu.sync_copy(dst_hbm.at[pl.ds(chunk_base + h * hh, hh)],
                                didx.at[pl.ds(0, hh)])
                gather(0, 0)
                lax.fori_loop(0, hh // 2, step, 0)

        @pl.when(c == 0)
        def _():
            run(s * ca, ha)

        @pl.when(c == 1)
        def _():
            run(NS * ca + s * cb, hb)

        plsc.subcore_barrier()
        pltpu.sync_copy(acc.at[pl.ds(base, RPS)], out_hbm.at[c, pl.ds(base, RPS)])

    return spmm


_GRID = 8
_RB = NPAD // _GRID


def _dinv_body(p_ref, o_ref):
    deg = p_ref[0, :, 0:1] + p_ref[1, :, 0:1] + 1.0
    o_ref[:] = lax.rsqrt(deg)


def _dinv(partials16):
    return pl.pallas_call(
        _dinv_body,
        out_shape=jax.ShapeDtypeStruct((NPAD, 1), jnp.float32),
    )(partials16)


def _mm_scale_body(x_ref, w_ref, v_ref, o_ref):
    o_ref[:] = jnp.dot(x_ref[:], w_ref[:], preferred_element_type=jnp.float32) * v_ref[:]


def _mm_scale(x, w, dinv):
    dn = w.shape[1]
    return pl.pallas_call(
        _mm_scale_body,
        grid=(_GRID,),
        in_specs=[
            pl.BlockSpec((_RB, x.shape[1]), lambda i: (i, 0)),
            pl.BlockSpec(w.shape, lambda i: (0, 0)),
            pl.BlockSpec((_RB, 1), lambda i: (i, 0)),
        ],
        out_specs=pl.BlockSpec((_RB, dn), lambda i: (i, 0)),
        out_shape=jax.ShapeDtypeStruct((NPAD, dn), jnp.float32),
    )(x, w, dinv)


def _layer_body(p0_ref, p1_ref, y_ref, v_ref, b_ref, w_ref, o_ref):
    t = (p0_ref[:] + p1_ref[:] + y_ref[:]) * v_ref[:] + b_ref[:]
    h = jnp.maximum(t, 0.0)
    o_ref[:] = jnp.dot(h, w_ref[:], preferred_element_type=jnp.float32) * v_ref[:]


def _layer(p0, p1, y, dinv, b, w):
    d, dn = w.shape
    return pl.pallas_call(
        _layer_body,
        grid=(_GRID,),
        in_specs=[
            pl.BlockSpec((_RB, d), lambda i: (i, 0)),
            pl.BlockSpec((_RB, d), lambda i: (i, 0)),
            pl.BlockSpec((_RB, d), lambda i: (i, 0)),
            pl.BlockSpec((_RB, 1), lambda i: (i, 0)),
            pl.BlockSpec((1, d), lambda i: (0, 0)),
            pl.BlockSpec((d, dn), lambda i: (0, 0)),
        ],
        out_specs=pl.BlockSpec((_RB, dn), lambda i: (i, 0)),
        out_shape=jax.ShapeDtypeStruct((NPAD, dn), jnp.float32),
    )(p0, p1, y, dinv, b, w)


def _final_body(p0_ref, p1_ref, y_ref, v_ref, b_ref, o_ref):
    o = (p0_ref[:] + p1_ref[:] + y_ref[:]) * v_ref[:] + b_ref[:]
    z = o - jnp.max(o, axis=1, keepdims=True)
    o_ref[:] = z - jnp.log(jnp.sum(jnp.exp(z), axis=1, keepdims=True))


def _final(p0, p1, y, dinv, b):
    d = y.shape[1]
    return pl.pallas_call(
        _final_body,
        grid=(_GRID,),
        in_specs=[
            pl.BlockSpec((_RB, d), lambda i: (i, 0)),
            pl.BlockSpec((_RB, d), lambda i: (i, 0)),
            pl.BlockSpec((_RB, d), lambda i: (i, 0)),
            pl.BlockSpec((_RB, 1), lambda i: (i, 0)),
            pl.BlockSpec((1, d), lambda i: (0, 0)),
        ],
        out_specs=pl.BlockSpec((_RB, d), lambda i: (i, 0)),
        out_shape=jax.ShapeDtypeStruct((NPAD, d), jnp.float32),
    )(p0, p1, y, dinv, b)


def kernel(x, edge_index, W1, b1, W2, b2, W3, b3):
    n = x.shape[0]
    e = edge_index.shape[1]
    ei = edge_index.astype(jnp.int32)

    K, CA, CB = 128, 128, 32
    tot_ch = NS * (CA + CB)
    pad = jnp.full((tot_ch * K - e,), n, jnp.int32)
    src = jnp.concatenate([ei[0], pad]).reshape(tot_ch, K)
    dst = jnp.concatenate([ei[1], pad]).reshape(tot_ch, K)

    spmm16 = _make_spmm(16, K, CA, CB)
    spmm128 = _make_spmm(128, K, CA, CB)
    spmm64 = _make_spmm(64, K, CA, CB)

    ones16 = jnp.zeros((NPAD, 16), jnp.float32).at[:n, 0].set(1.0)
    deg_p = spmm16(ones16, src, dst)
    dinv = _dinv(deg_p)
    xp = jnp.zeros((NPAD, x.shape[1]), jnp.float32).at[:n].set(x)
    y1 = _mm_scale(xp, W1, dinv)
    p1 = spmm128(y1, src, dst)
    y2 = _layer(p1[0], p1[1], y1, dinv, b1.reshape(1, -1), W2)
    p2 = spmm128(y2, src, dst)
    y3 = _layer(p2[0], p2[1], y2, dinv, b2.reshape(1, -1), W3)
    p3 = spmm64(y3, src, dst)
    out = _final(p3[0], p3[1], y3, dinv, b3.reshape(1, -1))
    return out[:n]
